# baseline (speedup 1.0000x reference)
import functools

import jax
import jax.numpy as jnp
import numpy as np
from jax import lax
from jax.experimental import pallas as pl
from jax.experimental.pallas import tpu as pltpu
from jax.experimental.pallas import tpu_sc as plsc

_N, _C, _RES, _F = 32, 192, 32, 4
_LH = _RES // _F
_HW = _LH * _LH
_NB = _F * _F
_K = 6
_R = _NB * _N
_OC = _C // 10 + 1
_OUTD = _HW * _OC
_HID = _K * _C + _HW
_NROWS = _N * _RES * _RES
_CP = 256
_CINP = _K * _CP + _HW

_SC_NC, _SC_NS = 2, 16
_NW = _SC_NC * _SC_NS
_NIDX = _R * _K
_RPW = _R // _NW

_CHUNK = 2048


def _diag_mask() -> np.ndarray:
    idx = np.arange(_CHUNK)
    rr = (idx % (_RES * _RES)) // (_LH * _RES)
    ss = (idx % _RES) // _LH
    m = np.zeros((_CHUNK, _NB), np.float32)
    m[idx, rr * _F + ss] = 1.0
    return m


_DIAG = _diag_mask()


def _scores_body(xr_ref, b_ref, dm_ref, t_ref):
    a = xr_ref[0].astype(jnp.bfloat16)
    b = b_ref[...].astype(jnp.bfloat16)
    t_mm = jnp.dot(a, b, preferred_element_type=jnp.float32)
    t_ref[0] = jnp.sum(t_mm * dm_ref[...], axis=1, keepdims=True)


def _run_scores(xrows, cwt):
    xr3 = xrows.reshape(_NROWS // _CHUNK, _CHUNK, _C)
    return pl.pallas_call(
        _scores_body,
        grid=(_NROWS // _CHUNK,),
        in_specs=[
            pl.BlockSpec((1, _CHUNK, _C), lambda i: (i, 0, 0)),
            pl.BlockSpec((_C, _NB), lambda i: (0, 0)),
            pl.BlockSpec((_CHUNK, _NB), lambda i: (0, 0)),
        ],
        out_specs=pl.BlockSpec((1, _CHUNK, 1), lambda i: (i, 0, 0)),
        out_shape=jax.ShapeDtypeStruct((_NROWS // _CHUNK, _CHUNK, 1), jnp.float32),
    )(xr3, cwt, jnp.asarray(_DIAG))


def _select_body(t_ref, mask_ref, gidx_ref):
    t = t_ref[...]
    e = jnp.exp(t)
    normed = e / jnp.sum(e, axis=1, keepdims=True)
    pos = lax.broadcasted_iota(jnp.int32, (_R, _HW), 1)
    work = normed
    mask = jnp.zeros((_R, _HW), jnp.float32)
    for _ in range(_K):
        m = jnp.max(work, axis=1, keepdims=True)
        is_max = work == m
        cand = jnp.where(is_max, pos, _HW)
        sel = jnp.min(cand, axis=1, keepdims=True)
        one = pos == sel
        mask = mask + one.astype(jnp.float32)
        work = jnp.where(one, -jnp.inf, work)
    mask_ref[...] = mask
    r = lax.broadcasted_iota(jnp.int32, (_R, 1), 0)
    n = r & (_N - 1)
    bi = r >> 5
    rr = bi >> 2
    ss = bi & 3
    base = n * (_RES * _RES) + rr * (_LH * _RES) + ss * _LH
    candp = jnp.where(mask > 0.5, pos, _HW)
    cols = []
    for _ in range(_K):
        p = jnp.min(candp, axis=1, keepdims=True)
        cols.append(base + (p >> 3) * _RES + (p & 7))
        candp = jnp.where(candp == p, _HW, candp)
    gidx_ref[...] = jnp.concatenate(cols, axis=1)


def _run_select(t):
    mask, gidx = pl.pallas_call(
        _select_body,
        out_shape=[
            jax.ShapeDtypeStruct((_R, _HW), jnp.float32),
            jax.ShapeDtypeStruct((_R, _K), jnp.int32),
        ],
    )(t)
    return mask, gidx


def _make_sc_gather():
    mesh = plsc.VectorSubcoreMesh(core_axis_name="c", subcore_axis_name="s")

    @functools.partial(
        pl.kernel,
        mesh=mesh,
        out_type=jax.ShapeDtypeStruct((_R, _CINP), jnp.float32),
        scratch_types=[
            pltpu.VMEM((_RPW * _K,), jnp.int32),
            pltpu.VMEM((_RPW * _K, _CP), jnp.float32),
            pltpu.VMEM((_RPW, _HW), jnp.float32),
            pltpu.SemaphoreType.DMA,
        ],
    )
    def gather_k(table_hbm, idxt_hbm, mask_hbm, out_hbm, idx_v, rows_v, mask_v, sem):
        wid = lax.axis_index("s") * _SC_NC + lax.axis_index("c")
        base = wid * _RPW
        for j in range(_K):
            pltpu.sync_copy(
                idxt_hbm.at[pl.ds(j * _R + base, _RPW)],
                idx_v.at[pl.ds(j * _RPW, _RPW)],
            )
        pltpu.sync_copy(mask_hbm.at[pl.ds(base, _RPW)], mask_v)
        pltpu.async_copy(table_hbm.at[idx_v], rows_v, sem).wait()
        for j in range(_K):
            pltpu.sync_copy(
                rows_v.at[pl.ds(j * _RPW, _RPW)],
                out_hbm.at[pl.ds(base, _RPW), pl.ds(j * _CP, _CP)],
            )
        pltpu.sync_copy(mask_v, out_hbm.at[pl.ds(base, _RPW), pl.ds(_K * _CP, _HW)])

    return gather_k


_sc_gather_impl = None


def _sc_gather(table, idxt, mask):
    global _sc_gather_impl
    if _sc_gather_impl is None:
        _sc_gather_impl = _make_sc_gather()
    return _sc_gather_impl(table, idxt, mask)


def _mlp_body(cin_ref, w1_ref, b1_ref, w2_ref, b2_ref, out_ref):
    cin = cin_ref[...].astype(jnp.bfloat16)
    w1 = w1_ref[...].astype(jnp.bfloat16)
    hid = lax.dot_general(
        cin, w1, (((1,), (1,)), ((), ())),
        preferred_element_type=jnp.float32,
    )
    hid = jnp.maximum(hid + b1_ref[...], 0.0).astype(jnp.bfloat16)
    w2 = w2_ref[...].astype(jnp.bfloat16)
    out_ref[...] = (
        lax.dot_general(
            hid, w2, (((1,), (1,)), ((), ())),
            preferred_element_type=jnp.float32,
        )
        + b2_ref[...]
    )


def _run_mlp(cin, W1p, b1, W2p, b2p):
    return pl.pallas_call(
        _mlp_body,
        out_shape=jax.ShapeDtypeStruct((_R, _OUTD), jnp.float32),
    )(cin, W1p, b1.reshape(1, _HID), W2p, b2p.reshape(1, _OUTD))


def _outconv_body(rec_ref, ow_ref, out_ref):
    rb = rec_ref[0].reshape(_N * _HW, _OC)
    o = jnp.dot(rb, ow_ref[...], preferred_element_type=jnp.float32)
    out_ref[...] = o.reshape(_N, 1, _LH, 1, _LH, _C)


def _run_outconv(rec, owt):
    rec4 = rec.reshape(_NB, _N, _HW, _OC)
    return pl.pallas_call(
        _outconv_body,
        grid=(_NB,),
        in_specs=[
            pl.BlockSpec((1, _N, _HW, _OC), lambda i: (i, 0, 0, 0)),
            pl.BlockSpec((_OC, _C), lambda i: (0, 0)),
        ],
        out_specs=pl.BlockSpec(
            (_N, 1, _LH, 1, _LH, _C),
            lambda i: (0, i // _F, 0, i % _F, 0, 0),
        ),
        out_shape=jax.ShapeDtypeStruct(
            (_N, _F, _LH, _F, _LH, _C), jnp.float32
        ),
    )(rec4, owt)


def kernel(x, conv_w, W1, b1, W2, b2, out_w):
    n, c, h, w = x.shape
    xrows = x.transpose(0, 2, 3, 1).reshape(_NROWS, _C)
    table = jnp.pad(xrows, ((0, 0), (0, _CP - _C)))

    t_diag = _run_scores(xrows, conv_w.T)
    t_sel = (
        t_diag.reshape(_N, _F, _LH, _F, _LH)
        .transpose(1, 3, 0, 2, 4)
        .reshape(_R, _HW)
    )
    mask, gidx = _run_select(t_sel)

    idxt = gidx.T.reshape(_NIDX)
    cin = _sc_gather(table, idxt, mask)

    W1p = jnp.concatenate(
        [
            jnp.pad(
                W1[:, : _K * _C].reshape(_HID, _K, _C),
                ((0, 0), (0, 0), (0, _CP - _C)),
            ).reshape(_HID, _K * _CP),
            W1[:, _K * _C :],
        ],
        axis=1,
    )
    W2p = W2.reshape(_OC, _HW, _HID).transpose(1, 0, 2).reshape(_OUTD, _HID)
    b2p = b2.reshape(_OC, _HW).T.reshape(_OUTD)

    rec = _run_mlp(cin, W1p, b1, W2p, b2p)
    out6 = _run_outconv(rec, out_w.T)
    return out6.reshape(_N, _RES, _RES, _C).transpose(0, 3, 1, 2)


# speedup vs baseline: 1.0952x; 1.0952x over previous
import functools

import jax
import jax.numpy as jnp
import numpy as np
from jax import lax
from jax.experimental import pallas as pl
from jax.experimental.pallas import tpu as pltpu
from jax.experimental.pallas import tpu_sc as plsc

_N, _C, _RES, _F = 32, 192, 32, 4
_LH = _RES // _F
_HW = _LH * _LH
_NB = _F * _F
_K = 6
_R = _NB * _N
_OC = _C // 10 + 1
_OUTD = _HW * _OC
_HID = _K * _C + _HW
_NROWS = _N * _RES * _RES
_CP = 256
_CINP = _K * _CP + _HW

_SC_NC, _SC_NS = 2, 16
_NW = _SC_NC * _SC_NS
_NIDX = _R * _K
_RPW = _R // _NW

_CHUNK = 2048


def _diag_mask() -> np.ndarray:
    idx = np.arange(_CHUNK)
    rr = (idx % (_RES * _RES)) // (_LH * _RES)
    ss = (idx % _RES) // _LH
    m = np.zeros((_CHUNK, _NB), np.float32)
    m[idx, rr * _F + ss] = 1.0
    return m


_DIAG = _diag_mask()


def _scores_body(xr_ref, b_ref, dm_ref, t_ref, tab_ref):
    a32 = xr_ref[0]
    a = a32.astype(jnp.bfloat16)
    b = b_ref[...].astype(jnp.bfloat16)
    t_mm = jnp.dot(a, b, preferred_element_type=jnp.float32)
    t_ref[0] = jnp.sum(t_mm * dm_ref[...], axis=1, keepdims=True)
    tab_ref[0] = jnp.pad(a32, ((0, 0), (0, _CP - _C)))


def _run_scores(xrows, cwt):
    xr3 = xrows.reshape(_NROWS // _CHUNK, _CHUNK, _C)
    return pl.pallas_call(
        _scores_body,
        grid=(_NROWS // _CHUNK,),
        in_specs=[
            pl.BlockSpec((1, _CHUNK, _C), lambda i: (i, 0, 0)),
            pl.BlockSpec((_C, _NB), lambda i: (0, 0)),
            pl.BlockSpec((_CHUNK, _NB), lambda i: (0, 0)),
        ],
        out_specs=[
            pl.BlockSpec((1, _CHUNK, 1), lambda i: (i, 0, 0)),
            pl.BlockSpec((1, _CHUNK, _CP), lambda i: (i, 0, 0)),
        ],
        out_shape=[
            jax.ShapeDtypeStruct((_NROWS // _CHUNK, _CHUNK, 1), jnp.float32),
            jax.ShapeDtypeStruct((_NROWS // _CHUNK, _CHUNK, _CP), jnp.float32),
        ],
    )(xr3, cwt, jnp.asarray(_DIAG))


def _select_body(t_ref, mask_ref, gidx_ref):
    t = t_ref[...]
    e = jnp.exp(t)
    normed = e / jnp.sum(e, axis=1, keepdims=True)
    pos = lax.broadcasted_iota(jnp.int32, (_R, _HW), 1)
    work = normed
    mask = jnp.zeros((_R, _HW), jnp.float32)
    for _ in range(_K):
        m = jnp.max(work, axis=1, keepdims=True)
        is_max = work == m
        cand = jnp.where(is_max, pos, _HW)
        sel = jnp.min(cand, axis=1, keepdims=True)
        one = pos == sel
        mask = mask + one.astype(jnp.float32)
        work = jnp.where(one, -jnp.inf, work)
    mask_ref[...] = mask
    r = lax.broadcasted_iota(jnp.int32, (_R, 1), 0)
    n = r & (_N - 1)
    bi = r >> 5
    rr = bi >> 2
    ss = bi & 3
    base = n * (_RES * _RES) + rr * (_LH * _RES) + ss * _LH
    candp = jnp.where(mask > 0.5, pos, _HW)
    cols = []
    for _ in range(_K):
        p = jnp.min(candp, axis=1, keepdims=True)
        cols.append(base + (p >> 3) * _RES + (p & 7))
        candp = jnp.where(candp == p, _HW, candp)
    gidx_ref[...] = jnp.concatenate(cols, axis=1)


def _run_select(t):
    mask, gidx = pl.pallas_call(
        _select_body,
        out_shape=[
            jax.ShapeDtypeStruct((_R, _HW), jnp.float32),
            jax.ShapeDtypeStruct((_R, _K), jnp.int32),
        ],
    )(t)
    return mask, gidx


def _make_sc_gather():
    mesh = plsc.VectorSubcoreMesh(core_axis_name="c", subcore_axis_name="s")

    @functools.partial(
        pl.kernel,
        mesh=mesh,
        out_type=jax.ShapeDtypeStruct((_R, _CINP), jnp.float32),
        scratch_types=[
            pltpu.VMEM((_RPW * _K,), jnp.int32),
            pltpu.VMEM((_RPW * _K, _CP), jnp.float32),
            pltpu.VMEM((_RPW, _HW), jnp.float32),
            pltpu.SemaphoreType.DMA,
        ],
    )
    def gather_k(table_hbm, idxt_hbm, mask_hbm, out_hbm, idx_v, rows_v, mask_v, sem):
        wid = lax.axis_index("s") * _SC_NC + lax.axis_index("c")
        base = wid * _RPW
        pltpu.sync_copy(idxt_hbm.at[pl.ds(wid * _RPW * _K, _RPW * _K)], idx_v)
        pltpu.sync_copy(mask_hbm.at[pl.ds(base, _RPW)], mask_v)
        pltpu.async_copy(table_hbm.at[idx_v], rows_v, sem).wait()
        for j in range(_K):
            pltpu.sync_copy(
                rows_v.at[pl.ds(j * _RPW, _RPW)],
                out_hbm.at[pl.ds(base, _RPW), pl.ds(j * _CP, _CP)],
            )
        pltpu.sync_copy(mask_v, out_hbm.at[pl.ds(base, _RPW), pl.ds(_K * _CP, _HW)])

    return gather_k


_sc_gather_impl = None


def _sc_gather(table, idxt, mask):
    global _sc_gather_impl
    if _sc_gather_impl is None:
        _sc_gather_impl = _make_sc_gather()
    return _sc_gather_impl(table, idxt, mask)


def _mlp_body(cin_ref, w1_ref, b1_ref, w2_ref, b2_ref, out_ref):
    cin = cin_ref[...].astype(jnp.bfloat16)
    w1 = w1_ref[...].astype(jnp.bfloat16)
    hid = lax.dot_general(
        cin, w1, (((1,), (1,)), ((), ())),
        preferred_element_type=jnp.float32,
    )
    hid = jnp.maximum(hid + b1_ref[...], 0.0).astype(jnp.bfloat16)
    w2 = w2_ref[...].astype(jnp.bfloat16)
    out_ref[...] = (
        lax.dot_general(
            hid, w2, (((1,), (1,)), ((), ())),
            preferred_element_type=jnp.float32,
        )
        + b2_ref[...]
    ).astype(jnp.bfloat16)


def _run_mlp(cin, W1p, b1, W2p, b2p):
    return pl.pallas_call(
        _mlp_body,
        out_shape=jax.ShapeDtypeStruct((_R, _OUTD), jnp.bfloat16),
    )(cin, W1p, b1.reshape(1, _HID), W2p, b2p.reshape(1, _OUTD))


def _outconv_body(rec_ref, ow_ref, out_ref):
    rb = rec_ref[0].reshape(_N * _HW, _OC)
    ow = ow_ref[...].astype(jnp.bfloat16)
    o = jnp.dot(rb, ow, preferred_element_type=jnp.float32)
    out_ref[...] = o.reshape(_N, 1, _LH, 1, _LH, _C)


def _run_outconv(rec, owt):
    rec4 = rec.reshape(_NB, _N, _HW, _OC)
    return pl.pallas_call(
        _outconv_body,
        grid=(_NB,),
        in_specs=[
            pl.BlockSpec((1, _N, _HW, _OC), lambda i: (i, 0, 0, 0)),
            pl.BlockSpec((_OC, _C), lambda i: (0, 0)),
        ],
        out_specs=pl.BlockSpec(
            (_N, 1, _LH, 1, _LH, _C),
            lambda i: (0, i // _F, 0, i % _F, 0, 0),
        ),
        out_shape=jax.ShapeDtypeStruct(
            (_N, _F, _LH, _F, _LH, _C), jnp.float32
        ),
    )(rec4, owt)


def kernel(x, conv_w, W1, b1, W2, b2, out_w):
    n, c, h, w = x.shape
    xrows = x.transpose(0, 2, 3, 1).reshape(_NROWS, _C)

    t_diag, table3 = _run_scores(xrows, conv_w.T)
    table = table3.reshape(_NROWS, _CP)
    t_sel = (
        t_diag.reshape(_N, _F, _LH, _F, _LH)
        .transpose(1, 3, 0, 2, 4)
        .reshape(_R, _HW)
    )
    mask, gidx = _run_select(t_sel)

    idxt = (
        gidx.reshape(_NW, _RPW, _K).transpose(0, 2, 1).reshape(_NIDX)
    )
    cin = _sc_gather(table, idxt, mask)

    W1p = jnp.concatenate(
        [
            jnp.pad(
                W1[:, : _K * _C].reshape(_HID, _K, _C),
                ((0, 0), (0, 0), (0, _CP - _C)),
            ).reshape(_HID, _K * _CP),
            W1[:, _K * _C :],
        ],
        axis=1,
    )
    W2p = W2.reshape(_OC, _HW, _HID).transpose(1, 0, 2).reshape(_OUTD, _HID)
    b2p = b2.reshape(_OC, _HW).T.reshape(_OUTD)

    rec = _run_mlp(cin, W1p, b1, W2p, b2p)
    out6 = _run_outconv(rec, out_w.T)
    return out6.reshape(_N, _RES, _RES, _C).transpose(0, 3, 1, 2)


# speedup vs baseline: 1.2886x; 1.1766x over previous
import functools

import jax
import jax.numpy as jnp
import numpy as np
from jax import lax
from jax.experimental import pallas as pl
from jax.experimental.pallas import tpu as pltpu
from jax.experimental.pallas import tpu_sc as plsc

_N, _C, _RES, _F = 32, 192, 32, 4
_LH = _RES // _F
_HW = _LH * _LH
_NB = _F * _F
_K = 6
_R = _NB * _N
_OC = _C // 10 + 1
_OUTD = _HW * _OC
_HID = _K * _C + _HW
_NROWS = _N * _RES * _RES
_CP = 256
_CINP = _K * _CP + _HW

_SC_NC, _SC_NS = 2, 16
_NW = _SC_NC * _SC_NS
_NIDX = _R * _K
_RPW = _R // _NW

_CHUNK = 2048


def _diag_mask() -> np.ndarray:
    idx = np.arange(_CHUNK)
    rr = (idx % (_RES * _RES)) // (_LH * _RES)
    ss = (idx % _RES) // _LH
    m = np.zeros((_CHUNK, _NB), np.float32)
    m[idx, rr * _F + ss] = 1.0
    return m


_DIAG = _diag_mask()


def _scores_body(xr_ref, b_ref, dm_ref, t_ref, tab_ref):
    a32 = xr_ref[0]
    a = a32.astype(jnp.bfloat16)
    b = b_ref[...].astype(jnp.bfloat16)
    t_mm = jnp.dot(a, b, preferred_element_type=jnp.float32)
    t_ref[0] = jnp.sum(t_mm * dm_ref[...], axis=1, keepdims=True)
    tab_ref[0] = jnp.pad(a32, ((0, 0), (0, _CP - _C)))


def _run_scores(xrows, cwt):
    xr3 = xrows.reshape(_NROWS // _CHUNK, _CHUNK, _C)
    return pl.pallas_call(
        _scores_body,
        grid=(_NROWS // _CHUNK,),
        in_specs=[
            pl.BlockSpec((1, _CHUNK, _C), lambda i: (i, 0, 0)),
            pl.BlockSpec((_C, _NB), lambda i: (0, 0)),
            pl.BlockSpec((_CHUNK, _NB), lambda i: (0, 0)),
        ],
        out_specs=[
            pl.BlockSpec((1, _CHUNK, 1), lambda i: (i, 0, 0)),
            pl.BlockSpec((1, _CHUNK, _CP), lambda i: (i, 0, 0)),
        ],
        out_shape=[
            jax.ShapeDtypeStruct((_NROWS // _CHUNK, _CHUNK, 1), jnp.float32),
            jax.ShapeDtypeStruct((_NROWS // _CHUNK, _CHUNK, _CP), jnp.float32),
        ],
    )(xr3, cwt, jnp.asarray(_DIAG))


def _select_body(t_ref, mask_ref, gidx_ref):
    t = t_ref[...]
    e = jnp.exp(t)
    normed = e / jnp.sum(e, axis=1, keepdims=True)
    pos = lax.broadcasted_iota(jnp.int32, (_R, _HW), 1)
    work = normed
    mask = jnp.zeros((_R, _HW), jnp.float32)
    for _ in range(_K):
        m = jnp.max(work, axis=1, keepdims=True)
        is_max = work == m
        cand = jnp.where(is_max, pos, _HW)
        sel = jnp.min(cand, axis=1, keepdims=True)
        one = pos == sel
        mask = mask + one.astype(jnp.float32)
        work = jnp.where(one, -jnp.inf, work)
    mask_ref[...] = mask
    r = lax.broadcasted_iota(jnp.int32, (_R, 1), 0)
    n = r & (_N - 1)
    bi = r >> 5
    rr = bi >> 2
    ss = bi & 3
    base = n * (_RES * _RES) + rr * (_LH * _RES) + ss * _LH
    candp = jnp.where(mask > 0.5, pos, _HW)
    cols = []
    for _ in range(_K):
        p = jnp.min(candp, axis=1, keepdims=True)
        cols.append(base + (p >> 3) * _RES + (p & 7))
        candp = jnp.where(candp == p, _HW, candp)
    gidx_ref[...] = jnp.concatenate(cols, axis=1)


def _run_select(t):
    mask, gidx = pl.pallas_call(
        _select_body,
        out_shape=[
            jax.ShapeDtypeStruct((_R, _HW), jnp.float32),
            jax.ShapeDtypeStruct((_R, _K), jnp.int32),
        ],
    )(t)
    return mask, gidx


def _make_sc_gather():
    mesh = plsc.VectorSubcoreMesh(core_axis_name="c", subcore_axis_name="s")

    @functools.partial(
        pl.kernel,
        mesh=mesh,
        out_type=jax.ShapeDtypeStruct((_R, _CINP), jnp.float32),
        scratch_types=[
            pltpu.VMEM((_RPW * _K,), jnp.int32),
            pltpu.VMEM((_RPW * _K, _CP), jnp.float32),
            pltpu.VMEM((_RPW, _HW), jnp.float32),
            pltpu.SemaphoreType.DMA,
        ],
    )
    def gather_k(table_hbm, idxt_hbm, mask_hbm, out_hbm, idx_v, rows_v, mask_v, sem):
        wid = lax.axis_index("s") * _SC_NC + lax.axis_index("c")
        base = wid * _RPW
        pltpu.sync_copy(idxt_hbm.at[pl.ds(wid * _RPW * _K, _RPW * _K)], idx_v)
        pltpu.sync_copy(mask_hbm.at[pl.ds(base, _RPW)], mask_v)
        pltpu.async_copy(table_hbm.at[idx_v], rows_v, sem).wait()
        for j in range(_K):
            pltpu.sync_copy(
                rows_v.at[pl.ds(j * _RPW, _RPW)],
                out_hbm.at[pl.ds(base, _RPW), pl.ds(j * _CP, _CP)],
            )
        pltpu.sync_copy(mask_v, out_hbm.at[pl.ds(base, _RPW), pl.ds(_K * _CP, _HW)])

    return gather_k


_sc_gather_impl = None


def _sc_gather(table, idxt, mask):
    global _sc_gather_impl
    if _sc_gather_impl is None:
        _sc_gather_impl = _make_sc_gather()
    return _sc_gather_impl(table, idxt, mask)


def _mlp_body(cin_ref, w1_ref, b1_ref, w2_ref, b2_ref, out_ref):
    cin = cin_ref[...].astype(jnp.bfloat16)
    hid = lax.dot_general(
        cin, w1_ref[...], (((1,), (1,)), ((), ())),
        preferred_element_type=jnp.float32,
    )
    hid = jnp.maximum(hid + b1_ref[...], 0.0).astype(jnp.bfloat16)
    rec = (
        lax.dot_general(
            hid, w2_ref[...], (((1,), (1,)), ((), ())),
            preferred_element_type=jnp.float32,
        )
        + b2_ref[...]
    ).astype(jnp.bfloat16)
    out_ref[...] = rec


def _run_mlp(cin, W1p, b1, W2p, b2p):
    return pl.pallas_call(
        _mlp_body,
        out_shape=jax.ShapeDtypeStruct((_R, _OUTD), jnp.bfloat16),
    )(cin, W1p, b1.reshape(1, _HID), W2p, b2p.reshape(1, _OUTD))


def _outconv_body(rec_ref, ow_ref, out_ref):
    rb = rec_ref[0].reshape(_N * _HW, _OC)
    ow = ow_ref[...].astype(jnp.bfloat16)
    o = jnp.dot(rb, ow, preferred_element_type=jnp.float32)
    out_ref[...] = o.reshape(_N, 1, _LH, 1, _LH, _C)


def _run_outconv(rec, owt):
    rec3 = rec.reshape(_NB, _N, _HW, _OC)
    return pl.pallas_call(
        _outconv_body,
        grid=(_NB,),
        in_specs=[
            pl.BlockSpec((1, _N, _HW, _OC), lambda i: (i, 0, 0, 0)),
            pl.BlockSpec((_OC, _C), lambda i: (0, 0)),
        ],
        out_specs=pl.BlockSpec(
            (_N, 1, _LH, 1, _LH, _C),
            lambda i: (0, i // _F, 0, i % _F, 0, 0),
        ),
        out_shape=jax.ShapeDtypeStruct(
            (_N, _F, _LH, _F, _LH, _C), jnp.float32
        ),
    )(rec3, owt)


def kernel(x, conv_w, W1, b1, W2, b2, out_w):
    n, c, h, w = x.shape
    xrows = x.transpose(0, 2, 3, 1).reshape(_NROWS, _C)

    t_diag, table3 = _run_scores(xrows, conv_w.T)
    table = table3.reshape(_NROWS, _CP)
    t_sel = (
        t_diag[:, :, 0]
        .reshape(_N, _F, _LH, _F, _LH)
        .transpose(1, 3, 0, 2, 4)
        .reshape(_R, _HW)
    )
    mask, gidx = _run_select(t_sel)

    idxt = (
        gidx.reshape(_NW, _RPW, _K).transpose(0, 2, 1).reshape(_NIDX)
    )
    cin = _sc_gather(table, idxt, mask)

    W1p = jnp.concatenate(
        [
            jnp.pad(
                W1[:, : _K * _C].reshape(_HID, _K, _C),
                ((0, 0), (0, 0), (0, _CP - _C)),
            ).reshape(_HID, _K * _CP),
            W1[:, _K * _C :],
        ],
        axis=1,
    ).astype(jnp.bfloat16)
    W2p = (
        W2.reshape(_OC, _HW, _HID)
        .transpose(1, 0, 2)
        .reshape(_OUTD, _HID)
        .astype(jnp.bfloat16)
    )
    b2p = b2.reshape(_OC, _HW).T.reshape(_OUTD)

    rec = _run_mlp(cin, W1p, b1, W2p, b2p)
    out6 = _run_outconv(rec, out_w.T)
    return out6.reshape(_N, _RES, _RES, _C).transpose(0, 3, 1, 2)
